# Initial kernel scaffold; baseline (speedup 1.0000x reference)
import jax
import jax.numpy as jnp
from jax.experimental import pallas as pl


def kernel(x, A, W_pool0, b_pool0, W_self0, W_pool1, b_pool1, W_self1, W_head, b_head):
    raise NotImplementedError("write your pallas kernel here")


# speedup vs baseline: 1.1855x; 1.1855x over previous
import functools

import jax
import jax.numpy as jnp
from jax import lax
from jax.experimental import pallas as pl
from jax.experimental.pallas import tpu as pltpu
from jax.experimental.pallas import tpu_sc as plsc

_N = 10000
_E = 320000
_H = 128
_NW = 32
_R = 320
_NPAD = _NW * _R
_G = 128
_C = 2000


def _agg_body(src_hbm, dst_hbm, m_hbm, agg_hbm,
              acc, dstc, srcc, srcbuf, dlbuf, rows, sem):
    wid = lax.axis_index("s") * 2 + lax.axis_index("c")
    lo = wid * _R

    def _zero(i, carry):
        for f in range(_H // 16):
            acc[i, pl.ds(f * 16, 16)] = jnp.zeros((16,), jnp.float32)
        return carry
    lax.fori_loop(0, _R + 1, _zero, 0)

    def _flush(cnt):
        pltpu.async_copy(m_hbm.at[srcbuf.at[pl.ds(0, _G)]], rows, sem).wait()

        def _acc_one(j, carry):
            dl = dlbuf[pl.ds(j, 16)][0]
            for f in range(_H // 16):
                sl = pl.ds(f * 16, 16)
                acc[dl, sl] = jnp.maximum(acc[dl, sl], rows[j, sl])
            return carry
        lax.fori_loop(0, _G, _acc_one, 0)
        return cnt

    def _group(g, cnt):
        d = dstc[pl.ds(g * 16, 16)]
        s = srcc[pl.ds(g * 16, 16)]
        msk = (d >= lo) & (d < lo + _R)
        inc = plsc.cumsum(jnp.where(msk, 1, 0).astype(jnp.int32))
        pos = cnt + inc - 1
        plsc.store_scatter(srcbuf, [pos], s, mask=msk)
        plsc.store_scatter(dlbuf, [pos], d - lo, mask=msk)
        cnt = cnt + jnp.max(inc)

        def _do_flush(c):
            c = _flush(c)
            sv = srcbuf[pl.ds(_G, 16)]
            dv = dlbuf[pl.ds(_G, 16)]
            srcbuf[pl.ds(0, 16)] = sv
            dlbuf[pl.ds(0, 16)] = dv
            return c - _G
        cnt = lax.cond(cnt >= _G, _do_flush, lambda c: c, cnt)
        return cnt

    def _chunk(c, cnt):
        pltpu.sync_copy(dst_hbm.at[pl.ds(c * _C, _C)], dstc)
        pltpu.sync_copy(src_hbm.at[pl.ds(c * _C, _C)], srcc)
        return lax.fori_loop(0, _C // 16, _group, cnt)

    cnt = lax.fori_loop(0, _E // _C, _chunk, jnp.int32(0))

    iota = lax.broadcasted_iota(jnp.int32, (16,), 0)
    def _pad(g, carry):
        p = g * 16 + iota
        mskp = p >= cnt
        sv = srcbuf[pl.ds(g * 16, 16)]
        dv = dlbuf[pl.ds(g * 16, 16)]
        srcbuf[pl.ds(g * 16, 16)] = jnp.where(mskp, 0, sv)
        dlbuf[pl.ds(g * 16, 16)] = jnp.where(mskp, _R, dv)
        return carry
    lax.fori_loop(0, _G // 16, _pad, 0)
    _flush(cnt)

    pltpu.sync_copy(acc.at[pl.ds(0, _R), :], agg_hbm.at[pl.ds(lo, _R), :])


@functools.partial(
    pl.kernel,
    out_type=jax.ShapeDtypeStruct((_NPAD, _H), jnp.float32),
    mesh=plsc.VectorSubcoreMesh(core_axis_name="c", subcore_axis_name="s"),
    compiler_params=pltpu.CompilerParams(needs_layout_passes=False),
    scratch_types=[
        pltpu.VMEM((_R + 1, _H), jnp.float32),
        pltpu.VMEM((_C,), jnp.int32),
        pltpu.VMEM((_C,), jnp.int32),
        pltpu.VMEM((_G + 16,), jnp.int32),
        pltpu.VMEM((_G + 16,), jnp.int32),
        pltpu.VMEM((_G, _H), jnp.float32),
        pltpu.SemaphoreType.DMA,
    ],
)
def _segment_max(src_hbm, dst_hbm, m_hbm, agg_hbm,
                 acc, dstc, srcc, srcbuf, dlbuf, rows, sem):
    _agg_body(src_hbm, dst_hbm, m_hbm, agg_hbm,
              acc, dstc, srcc, srcbuf, dlbuf, rows, sem)


_BLK = 2000


def _tc0_body(x_ref, wp_ref, b_ref, ws_ref, m_ref, s_ref):
    x = x_ref[...]
    m = jnp.dot(x, wp_ref[...], preferred_element_type=jnp.float32) + b_ref[...]
    m_ref[...] = jnp.maximum(m, 0.0)
    s_ref[...] = jnp.dot(x, ws_ref[...], preferred_element_type=jnp.float32)


def _tc1_body(s_ref, a_ref, wp_ref, b_ref, ws_ref, m_ref, so_ref):
    ha = jnp.maximum(s_ref[...], 0.0)
    hb = jnp.maximum(a_ref[...], 0.0)
    ss = (jnp.sum(ha * ha, axis=1, keepdims=True)
          + jnp.sum(hb * hb, axis=1, keepdims=True))
    inv = 1.0 / jnp.maximum(jnp.sqrt(ss), 1e-12)
    ha = ha * inv
    hb = hb * inv
    wp = wp_ref[...]
    m = (jnp.dot(ha, wp[:_H], preferred_element_type=jnp.float32)
         + jnp.dot(hb, wp[_H:], preferred_element_type=jnp.float32)
         + b_ref[...])
    m_ref[...] = jnp.maximum(m, 0.0)
    ws = ws_ref[...]
    so_ref[...] = (jnp.dot(ha, ws[:_H], preferred_element_type=jnp.float32)
                   + jnp.dot(hb, ws[_H:], preferred_element_type=jnp.float32))


def _head_body(s_ref, a_ref, wh_ref, b_ref, o_ref):
    ha = jnp.maximum(s_ref[...], 0.0)
    hb = jnp.maximum(a_ref[...], 0.0)
    ss = (jnp.sum(ha * ha, axis=1, keepdims=True)
          + jnp.sum(hb * hb, axis=1, keepdims=True))
    inv = 1.0 / jnp.maximum(jnp.sqrt(ss), 1e-12)
    ha = ha * inv
    hb = hb * inv
    wh = wh_ref[...]
    o_ref[...] = (jnp.dot(ha, wh[:_H], preferred_element_type=jnp.float32)
                  + jnp.dot(hb, wh[_H:], preferred_element_type=jnp.float32)
                  + b_ref[...])


def _full(shape):
    return pl.BlockSpec(shape, lambda i: (0,) * len(shape))


def _rows(w):
    return pl.BlockSpec((_BLK, w), lambda i: (i, 0))


def _tc0(x, wp, b, ws):
    return pl.pallas_call(
        _tc0_body,
        grid=(_N // _BLK,),
        in_specs=[_rows(128), _full((128, _H)), _full((1, _H)), _full((128, _H))],
        out_specs=[_rows(_H), _rows(_H)],
        out_shape=[jax.ShapeDtypeStruct((_N, _H), jnp.float32)] * 2,
    )(x, wp, b.reshape(1, _H), ws)


def _tc1(s, a, wp, b, ws):
    return pl.pallas_call(
        _tc1_body,
        grid=(_N // _BLK,),
        in_specs=[_rows(_H), _rows(_H), _full((2 * _H, _H)), _full((1, _H)),
                  _full((2 * _H, _H))],
        out_specs=[_rows(_H), _rows(_H)],
        out_shape=[jax.ShapeDtypeStruct((_N, _H), jnp.float32)] * 2,
    )(s, a, wp, b.reshape(1, _H), ws)


def _head(s, a, wh, b):
    c = wh.shape[1]
    return pl.pallas_call(
        _head_body,
        grid=(_N // _BLK,),
        in_specs=[_rows(_H), _rows(_H), _full((2 * _H, c)), _full((1, c))],
        out_specs=_rows(c),
        out_shape=jax.ShapeDtypeStruct((_N, c), jnp.float32),
    )(s, a, wh, b.reshape(1, c))


def kernel(x, A, W_pool0, b_pool0, W_self0, W_pool1, b_pool1, W_self1,
           W_head, b_head):
    src = A[0]
    dst = A[1]
    m0, s0 = _tc0(x, W_pool0, b_pool0, W_self0)
    agg0 = _segment_max(src, dst, m0)[:_N]
    m1, s1 = _tc1(s0, agg0, W_pool1, b_pool1, W_self1)
    agg1 = _segment_max(src, dst, m1)[:_N]
    return _head(s1, agg1, W_head, b_head)


# speedup vs baseline: 2.3541x; 1.9858x over previous
import functools

import jax
import jax.numpy as jnp
from jax import lax
from jax.experimental import pallas as pl
from jax.experimental.pallas import tpu as pltpu
from jax.experimental.pallas import tpu_sc as plsc

_N = 10000
_E = 320000
_H = 128
_NW = 32
_R = 320
_NPAD = _NW * _R
_G = 128
_C = 2000
_NCH = _E // _C
_F = 2048
_CAP = 160 * _F
_BUF = 4608

_SC_PARAMS = pltpu.CompilerParams(needs_layout_passes=False)
_MESH = plsc.VectorSubcoreMesh(core_axis_name="c", subcore_axis_name="s")


def _bin_body(src_hbm, dst_hbm, ls_hbm, ld_hbm, cnts_hbm,
              dc0, dc1, sc0, sc1, srcbuf, dlbuf, cbuf,
              semd0, semd1, sems0, sems1):
    wid = lax.axis_index("s") * 2 + lax.axis_index("c")
    lo = wid * _R
    rbase = wid * _CAP
    dc, sc = (dc0, dc1), (sc0, sc1)
    semd, sems = (semd0, semd1), (sems0, sems1)

    def _stage(c, k):
        pltpu.async_copy(dst_hbm.at[pl.ds(c * _C, _C)], dc[k], semd[k])
        pltpu.async_copy(src_hbm.at[pl.ds(c * _C, _C)], sc[k], sems[k])

    def _wait(k):
        pltpu.make_async_copy(dst_hbm.at[pl.ds(0, _C)], dc[k], semd[k]).wait()
        pltpu.make_async_copy(src_hbm.at[pl.ds(0, _C)], sc[k], sems[k]).wait()

    _stage(0, 0)

    def _flush(cnt, nf):
        pltpu.sync_copy(srcbuf.at[pl.ds(0, _F)],
                        ls_hbm.at[pl.ds(rbase + nf * _F, _F)])
        pltpu.sync_copy(dlbuf.at[pl.ds(0, _F)],
                        ld_hbm.at[pl.ds(rbase + nf * _F, _F)])
        for k in range(125):
            srcbuf[pl.ds(k * 16, 16)] = srcbuf[pl.ds(_F + k * 16, 16)]
            dlbuf[pl.ds(k * 16, 16)] = dlbuf[pl.ds(_F + k * 16, 16)]
        return cnt - _F, nf + 1

    def _scan_chunk(dcr, scr, cnt):
        def _group(g, cnt):
            d = dcr[pl.ds(g * 16, 16)]
            s = scr[pl.ds(g * 16, 16)]
            dl = d - lo
            msk = plsc.bitcast(dl, jnp.uint32) < jnp.uint32(_R)
            plsc.store_compressed(srcbuf.at[pl.ds(cnt, 16)], s, mask=msk)
            plsc.store_compressed(dlbuf.at[pl.ds(cnt, 16)], dl, mask=msk)
            pc = plsc.all_reduce_population_count(msk)
            return cnt + pc[0]
        return lax.fori_loop(0, _C // 16, _group, cnt)

    def _chunk(i, carry):
        cnt, nf = carry
        for k in range(2):
            c = 2 * i + k

            @pl.when(c + 1 < _NCH)
            def _():
                _stage(c + 1, k ^ 1)
            _wait(k)
            cnt = _scan_chunk(dc[k], sc[k], cnt)
            cnt, nf = lax.cond(cnt >= _F, _flush,
                               lambda c_, n_: (c_, n_), cnt, nf)
        return cnt, nf

    cnt, nf = lax.fori_loop(0, _NCH // 2, _chunk,
                            (jnp.int32(0), jnp.int32(0)))

    iota = lax.broadcasted_iota(jnp.int32, (16,), 0)
    base = (cnt // 16) * 16
    for k in range(2 * _G // 16 + 2):
        at = base + k * 16
        pos = at + iota
        mp = pos >= cnt
        sv = srcbuf[pl.ds(at, 16)]
        dv = dlbuf[pl.ds(at, 16)]
        srcbuf[pl.ds(at, 16)] = jnp.where(mp, 0, sv)
        dlbuf[pl.ds(at, 16)] = jnp.where(mp, _R, dv)

    for blk in range(2):
        pltpu.sync_copy(srcbuf.at[pl.ds(blk * _F, _F)],
                        ls_hbm.at[pl.ds(rbase + (nf + blk) * _F, _F)])
        pltpu.sync_copy(dlbuf.at[pl.ds(blk * _F, _F)],
                        ld_hbm.at[pl.ds(rbase + (nf + blk) * _F, _F)])
    cbuf[pl.ds(0, 16)] = jnp.full((16,), nf * _F + cnt, jnp.int32)
    pltpu.sync_copy(cbuf, cnts_hbm.at[pl.ds(wid * 16, 16)])


@functools.partial(
    pl.kernel,
    out_type=(jax.ShapeDtypeStruct((_NW * _CAP,), jnp.int32),
              jax.ShapeDtypeStruct((_NW * _CAP,), jnp.int32),
              jax.ShapeDtypeStruct((_NW * 16,), jnp.int32)),
    mesh=_MESH,
    compiler_params=_SC_PARAMS,
    scratch_types=[
        pltpu.VMEM((_C,), jnp.int32),
        pltpu.VMEM((_C,), jnp.int32),
        pltpu.VMEM((_C,), jnp.int32),
        pltpu.VMEM((_C,), jnp.int32),
        pltpu.VMEM((_BUF,), jnp.int32),
        pltpu.VMEM((_BUF,), jnp.int32),
        pltpu.VMEM((16,), jnp.int32),
        pltpu.SemaphoreType.DMA,
        pltpu.SemaphoreType.DMA,
        pltpu.SemaphoreType.DMA,
        pltpu.SemaphoreType.DMA,
    ],
)
def _bin(src_hbm, dst_hbm, ls_hbm, ld_hbm, cnts_hbm, *rest):
    _bin_body(src_hbm, dst_hbm, ls_hbm, ld_hbm, cnts_hbm, *rest)


def _acc_body(ls_hbm, ld_hbm, cnts_hbm, m_hbm, agg_hbm,
              acc, si0, si1, db0, db1, r0, r1, cbuf, sg0, sg1):
    wid = lax.axis_index("s") * 2 + lax.axis_index("c")
    lo = wid * _R
    rbase = wid * _CAP
    si, db, rows, sg = (si0, si1), (db0, db1), (r0, r1), (sg0, sg1)

    pltpu.sync_copy(cnts_hbm.at[pl.ds(wid * 16, 16)], cbuf)
    cnt = cbuf[pl.ds(0, 16)][0]
    nb = jnp.maximum((cnt + _G - 1) // _G, 1)

    zeros = jnp.zeros((16,), jnp.float32)

    def _zero(i, carry):
        for f in range(_H // 16):
            acc[i, pl.ds(f * 16, 16)] = zeros
        return carry
    lax.fori_loop(0, _R + 1, _zero, 0)

    def _stage(b, k):
        pltpu.sync_copy(ls_hbm.at[pl.ds(rbase + b * _G, _G)], si[k])
        pltpu.sync_copy(ld_hbm.at[pl.ds(rbase + b * _G, _G)],
                        db[k].at[pl.ds(0, _G)])
        pltpu.async_copy(m_hbm.at[si[k]], rows[k], sg[k])

    _stage(0, 0)

    def _outer(i, carry):
        for k in range(2):
            b = 2 * i + k

            @pl.when(b < nb)
            def _():
                @pl.when(b + 1 < nb)
                def _():
                    _stage(b + 1, k ^ 1)
                pltpu.make_async_copy(m_hbm.at[si[k]], rows[k], sg[k]).wait()

                def _edge(j, carry2):
                    dl = db[k][pl.ds(j, 16)][0]
                    for f in range(_H // 16):
                        sl = pl.ds(f * 16, 16)
                        acc[dl, sl] = jnp.maximum(acc[dl, sl], rows[k][j, sl])
                    return carry2
                lax.fori_loop(0, _G, _edge, 0, unroll=2)
        return carry

    lax.fori_loop(0, (nb + 1) // 2, _outer, 0)
    pltpu.sync_copy(acc.at[pl.ds(0, _R), :], agg_hbm.at[pl.ds(lo, _R), :])


@functools.partial(
    pl.kernel,
    out_type=jax.ShapeDtypeStruct((_NPAD, _H), jnp.float32),
    mesh=_MESH,
    compiler_params=_SC_PARAMS,
    scratch_types=[
        pltpu.VMEM((_R + 1, _H), jnp.float32),
        pltpu.VMEM((_G,), jnp.int32),
        pltpu.VMEM((_G,), jnp.int32),
        pltpu.VMEM((_G + 16,), jnp.int32),
        pltpu.VMEM((_G + 16,), jnp.int32),
        pltpu.VMEM((_G, _H), jnp.float32),
        pltpu.VMEM((_G, _H), jnp.float32),
        pltpu.VMEM((16,), jnp.int32),
        pltpu.SemaphoreType.DMA,
        pltpu.SemaphoreType.DMA,
    ],
)
def _acc(ls_hbm, ld_hbm, cnts_hbm, m_hbm, agg_hbm, *rest):
    _acc_body(ls_hbm, ld_hbm, cnts_hbm, m_hbm, agg_hbm, *rest)


_BLK = 2000


def _tc0_body(x_ref, wp_ref, b_ref, ws_ref, m_ref, s_ref):
    x = x_ref[...]
    m = jnp.dot(x, wp_ref[...], preferred_element_type=jnp.float32) + b_ref[...]
    m_ref[...] = jnp.maximum(m, 0.0)
    s_ref[...] = jnp.dot(x, ws_ref[...], preferred_element_type=jnp.float32)


def _tc1_body(s_ref, a_ref, wp_ref, b_ref, ws_ref, m_ref, so_ref):
    ha = jnp.maximum(s_ref[...], 0.0)
    hb = jnp.maximum(a_ref[...], 0.0)
    ss = (jnp.sum(ha * ha, axis=1, keepdims=True)
          + jnp.sum(hb * hb, axis=1, keepdims=True))
    inv = 1.0 / jnp.maximum(jnp.sqrt(ss), 1e-12)
    ha = ha * inv
    hb = hb * inv
    wp = wp_ref[...]
    m = (jnp.dot(ha, wp[:_H], preferred_element_type=jnp.float32)
         + jnp.dot(hb, wp[_H:], preferred_element_type=jnp.float32)
         + b_ref[...])
    m_ref[...] = jnp.maximum(m, 0.0)
    ws = ws_ref[...]
    so_ref[...] = (jnp.dot(ha, ws[:_H], preferred_element_type=jnp.float32)
                   + jnp.dot(hb, ws[_H:], preferred_element_type=jnp.float32))


def _head_body(s_ref, a_ref, wh_ref, b_ref, o_ref):
    ha = jnp.maximum(s_ref[...], 0.0)
    hb = jnp.maximum(a_ref[...], 0.0)
    ss = (jnp.sum(ha * ha, axis=1, keepdims=True)
          + jnp.sum(hb * hb, axis=1, keepdims=True))
    inv = 1.0 / jnp.maximum(jnp.sqrt(ss), 1e-12)
    ha = ha * inv
    hb = hb * inv
    wh = wh_ref[...]
    o_ref[...] = (jnp.dot(ha, wh[:_H], preferred_element_type=jnp.float32)
                  + jnp.dot(hb, wh[_H:], preferred_element_type=jnp.float32)
                  + b_ref[...])


def _full(shape):
    return pl.BlockSpec(shape, lambda i: (0,) * len(shape))


def _rows(w):
    return pl.BlockSpec((_BLK, w), lambda i: (i, 0))


def _tc0(x, wp, b, ws):
    return pl.pallas_call(
        _tc0_body,
        grid=(_N // _BLK,),
        in_specs=[_rows(128), _full((128, _H)), _full((1, _H)), _full((128, _H))],
        out_specs=[_rows(_H), _rows(_H)],
        out_shape=[jax.ShapeDtypeStruct((_N, _H), jnp.float32)] * 2,
    )(x, wp, b.reshape(1, _H), ws)


def _tc1(s, a, wp, b, ws):
    return pl.pallas_call(
        _tc1_body,
        grid=(_N // _BLK,),
        in_specs=[_rows(_H), _rows(_H), _full((2 * _H, _H)), _full((1, _H)),
                  _full((2 * _H, _H))],
        out_specs=[_rows(_H), _rows(_H)],
        out_shape=[jax.ShapeDtypeStruct((_N, _H), jnp.float32)] * 2,
    )(s, a, wp, b.reshape(1, _H), ws)


def _head(s, a, wh, b):
    c = wh.shape[1]
    return pl.pallas_call(
        _head_body,
        grid=(_N // _BLK,),
        in_specs=[_rows(_H), _rows(_H), _full((2 * _H, c)), _full((1, c))],
        out_specs=_rows(c),
        out_shape=jax.ShapeDtypeStruct((_N, c), jnp.float32),
    )(s, a, wh, b.reshape(1, c))


def kernel(x, A, W_pool0, b_pool0, W_self0, W_pool1, b_pool1, W_self1,
           W_head, b_head):
    src = A[0]
    dst = A[1]
    ls, ld, cnts = _bin(src, dst)
    m0, s0 = _tc0(x, W_pool0, b_pool0, W_self0)
    agg0 = _acc(ls, ld, cnts, m0)[:_N]
    m1, s1 = _tc1(s0, agg0, W_pool1, b_pool1, W_self1)
    agg1 = _acc(ls, ld, cnts, m1)[:_N]
    return _head(s1, agg1, W_head, b_head)


# speedup vs baseline: 4.3526x; 1.8489x over previous
import functools

import jax
import jax.numpy as jnp
from jax import lax
from jax.experimental import pallas as pl
from jax.experimental.pallas import tpu as pltpu
from jax.experimental.pallas import tpu_sc as plsc

_N = 10000
_E = 320000
_H = 128
_NW = 32
_R = 320
_NPAD = _NW * _R
_G = 256
_C = 3200
_NCH = _E // _C
_F = 2048
_CAP = 160 * _F
_BUF = 5760

_SC_PARAMS = pltpu.CompilerParams(needs_layout_passes=False)
_MESH = plsc.VectorSubcoreMesh(core_axis_name="c", subcore_axis_name="s")


def _bin_body(src_hbm, dst_hbm, ls_hbm, ld_hbm, cnts_hbm,
              dc0, dc1, sc0, sc1, srcbuf, dlbuf, cbuf,
              semd0, semd1, sems0, sems1):
    wid = lax.axis_index("s") * 2 + lax.axis_index("c")
    lo = wid * _R
    rbase = wid * _CAP
    dc, sc = (dc0, dc1), (sc0, sc1)
    semd, sems = (semd0, semd1), (sems0, sems1)

    def _stage(c, k):
        pltpu.async_copy(dst_hbm.at[pl.ds(c * _C, _C)], dc[k], semd[k])
        pltpu.async_copy(src_hbm.at[pl.ds(c * _C, _C)], sc[k], sems[k])

    def _wait(k):
        pltpu.make_async_copy(dst_hbm.at[pl.ds(0, _C)], dc[k], semd[k]).wait()
        pltpu.make_async_copy(src_hbm.at[pl.ds(0, _C)], sc[k], sems[k]).wait()

    _stage(0, 0)

    def _flush(cnt, nf):
        pltpu.sync_copy(srcbuf.at[pl.ds(0, _F)],
                        ls_hbm.at[pl.ds(rbase + nf * _F, _F)])
        pltpu.sync_copy(dlbuf.at[pl.ds(0, _F)],
                        ld_hbm.at[pl.ds(rbase + nf * _F, _F)])
        for k in range(_C // 16):
            srcbuf[pl.ds(k * 16, 16)] = srcbuf[pl.ds(_F + k * 16, 16)]
            dlbuf[pl.ds(k * 16, 16)] = dlbuf[pl.ds(_F + k * 16, 16)]
        return cnt - _F, nf + 1

    def _scan_chunk(dcr, scr, cnt):
        def _quad(q, cnt):
            ss_, ds_, ms_, cs_ = [], [], [], []
            for u in range(4):
                g = q * 4 + u
                d = dcr[pl.ds(g * 16, 16)]
                s = scr[pl.ds(g * 16, 16)]
                dl = d - lo
                msk = plsc.bitcast(dl, jnp.uint32) < jnp.uint32(_R)
                pc = plsc.all_reduce_population_count(msk)
                ss_.append(s)
                ds_.append(dl)
                ms_.append(msk)
                cs_.append(pc[0])
            for u in range(4):
                plsc.store_compressed(srcbuf.at[pl.ds(cnt, 16)], ss_[u],
                                      mask=ms_[u])
                plsc.store_compressed(dlbuf.at[pl.ds(cnt, 16)], ds_[u],
                                      mask=ms_[u])
                cnt = cnt + cs_[u]
            return cnt
        return lax.fori_loop(0, _C // 64, _quad, cnt)

    def _chunk(i, carry):
        cnt, nf = carry
        for k in range(2):
            c = 2 * i + k

            @pl.when(c + 1 < _NCH)
            def _():
                _stage(c + 1, k ^ 1)
            _wait(k)
            cnt = _scan_chunk(dc[k], sc[k], cnt)
            for _ in range(2):
                cnt, nf = lax.cond(cnt >= _F, _flush,
                                   lambda c_, n_: (c_, n_), cnt, nf)
        return cnt, nf

    cnt, nf = lax.fori_loop(0, _NCH // 2, _chunk,
                            (jnp.int32(0), jnp.int32(0)))

    iota = lax.broadcasted_iota(jnp.int32, (16,), 0)
    base = (cnt // 16) * 16
    for k in range(2 * _G // 16 + 2):
        at = base + k * 16
        pos = at + iota
        mp = pos >= cnt
        sv = srcbuf[pl.ds(at, 16)]
        dv = dlbuf[pl.ds(at, 16)]
        srcbuf[pl.ds(at, 16)] = jnp.where(mp, 0, sv)
        dlbuf[pl.ds(at, 16)] = jnp.where(mp, _R, dv)

    for blk in range(2):
        pltpu.sync_copy(srcbuf.at[pl.ds(blk * _F, _F)],
                        ls_hbm.at[pl.ds(rbase + (nf + blk) * _F, _F)])
        pltpu.sync_copy(dlbuf.at[pl.ds(blk * _F, _F)],
                        ld_hbm.at[pl.ds(rbase + (nf + blk) * _F, _F)])
    cbuf[pl.ds(0, 16)] = jnp.full((16,), nf * _F + cnt, jnp.int32)
    pltpu.sync_copy(cbuf, cnts_hbm.at[pl.ds(wid * 16, 16)])


@functools.partial(
    pl.kernel,
    out_type=(jax.ShapeDtypeStruct((_NW * _CAP,), jnp.int32),
              jax.ShapeDtypeStruct((_NW * _CAP,), jnp.int32),
              jax.ShapeDtypeStruct((_NW * 16,), jnp.int32)),
    mesh=_MESH,
    compiler_params=_SC_PARAMS,
    scratch_types=[
        pltpu.VMEM((_C,), jnp.int32),
        pltpu.VMEM((_C,), jnp.int32),
        pltpu.VMEM((_C,), jnp.int32),
        pltpu.VMEM((_C,), jnp.int32),
        pltpu.VMEM((_BUF,), jnp.int32),
        pltpu.VMEM((_BUF,), jnp.int32),
        pltpu.VMEM((16,), jnp.int32),
        pltpu.SemaphoreType.DMA,
        pltpu.SemaphoreType.DMA,
        pltpu.SemaphoreType.DMA,
        pltpu.SemaphoreType.DMA,
    ],
)
def _bin(src_hbm, dst_hbm, ls_hbm, ld_hbm, cnts_hbm, *rest):
    _bin_body(src_hbm, dst_hbm, ls_hbm, ld_hbm, cnts_hbm, *rest)


def _acc_body(ls_hbm, ld_hbm, cnts_hbm, m_hbm, agg_hbm,
              acc, si0, si1, db0, db1, r0, r1, cbuf, sg0, sg1):
    wid = lax.axis_index("s") * 2 + lax.axis_index("c")
    lo = wid * _R
    rbase = wid * _CAP
    si, db, rows, sg = (si0, si1), (db0, db1), (r0, r1), (sg0, sg1)

    pltpu.sync_copy(cnts_hbm.at[pl.ds(wid * 16, 16)], cbuf)
    cnt = cbuf[pl.ds(0, 16)][0]
    nb = jnp.maximum((cnt + _G - 1) // _G, 1)

    zeros = jnp.zeros((16,), jnp.float32)

    def _zero(i, carry):
        for f in range(_H // 16):
            acc[i, pl.ds(f * 16, 16)] = zeros
        return carry
    lax.fori_loop(0, _R + 1, _zero, 0)

    def _stage(b, k):
        pltpu.sync_copy(ls_hbm.at[pl.ds(rbase + b * _G, _G)], si[k])
        pltpu.sync_copy(ld_hbm.at[pl.ds(rbase + b * _G, _G)], db[k])
        for h in range(2):
            pltpu.async_copy(m_hbm.at[si[k].at[pl.ds(h * 128, 128)]],
                             rows[k].at[pl.ds(h * 128, 128), :], sg[k])

    def _wait_rows(k):
        for h in range(2):
            pltpu.make_async_copy(m_hbm.at[si[k].at[pl.ds(h * 128, 128)]],
                                  rows[k].at[pl.ds(h * 128, 128), :],
                                  sg[k]).wait()

    _stage(0, 0)

    def _outer(i, carry):
        for k in range(2):
            b = 2 * i + k

            @pl.when(b < nb)
            def _():
                @pl.when(b + 1 < nb)
                def _():
                    _stage(b + 1, k ^ 1)
                _wait_rows(k)

                def _block(jb, carry2):
                    dlv = db[k][pl.ds(jb * 16, 16)]
                    for i16 in range(16):
                        j = jb * 16 + i16
                        dl = dlv[i16]
                        av = [acc[dl, pl.ds(f * 16, 16)]
                              for f in range(_H // 16)]
                        rv = [rows[k][j, pl.ds(f * 16, 16)]
                              for f in range(_H // 16)]
                        for f in range(_H // 16):
                            acc[dl, pl.ds(f * 16, 16)] = jnp.maximum(av[f],
                                                                     rv[f])
                    return carry2
                lax.fori_loop(0, _G // 16, _block, 0)
        return carry

    lax.fori_loop(0, (nb + 1) // 2, _outer, 0)
    pltpu.sync_copy(acc.at[pl.ds(0, _R), :], agg_hbm.at[pl.ds(lo, _R), :])


@functools.partial(
    pl.kernel,
    out_type=jax.ShapeDtypeStruct((_NPAD, _H), jnp.float32),
    mesh=_MESH,
    compiler_params=_SC_PARAMS,
    scratch_types=[
        pltpu.VMEM((_R + 1, _H), jnp.float32),
        pltpu.VMEM((_G,), jnp.int32),
        pltpu.VMEM((_G,), jnp.int32),
        pltpu.VMEM((_G,), jnp.int32),
        pltpu.VMEM((_G,), jnp.int32),
        pltpu.VMEM((_G, _H), jnp.float32),
        pltpu.VMEM((_G, _H), jnp.float32),
        pltpu.VMEM((16,), jnp.int32),
        pltpu.SemaphoreType.DMA,
        pltpu.SemaphoreType.DMA,
    ],
)
def _acc(ls_hbm, ld_hbm, cnts_hbm, m_hbm, agg_hbm, *rest):
    _acc_body(ls_hbm, ld_hbm, cnts_hbm, m_hbm, agg_hbm, *rest)


_BLK = 2000


def _tc0_body(x_ref, wp_ref, b_ref, ws_ref, m_ref, s_ref):
    x = x_ref[...]
    m = jnp.dot(x, wp_ref[...], preferred_element_type=jnp.float32) + b_ref[...]
    m_ref[...] = jnp.maximum(m, 0.0)
    s_ref[...] = jnp.dot(x, ws_ref[...], preferred_element_type=jnp.float32)


def _tc1_body(s_ref, a_ref, wp_ref, b_ref, ws_ref, m_ref, so_ref):
    ha = jnp.maximum(s_ref[...], 0.0)
    hb = jnp.maximum(a_ref[...], 0.0)
    ss = (jnp.sum(ha * ha, axis=1, keepdims=True)
          + jnp.sum(hb * hb, axis=1, keepdims=True))
    inv = 1.0 / jnp.maximum(jnp.sqrt(ss), 1e-12)
    ha = ha * inv
    hb = hb * inv
    wp = wp_ref[...]
    m = (jnp.dot(ha, wp[:_H], preferred_element_type=jnp.float32)
         + jnp.dot(hb, wp[_H:], preferred_element_type=jnp.float32)
         + b_ref[...])
    m_ref[...] = jnp.maximum(m, 0.0)
    ws = ws_ref[...]
    so_ref[...] = (jnp.dot(ha, ws[:_H], preferred_element_type=jnp.float32)
                   + jnp.dot(hb, ws[_H:], preferred_element_type=jnp.float32))


def _head_body(s_ref, a_ref, wh_ref, b_ref, o_ref):
    ha = jnp.maximum(s_ref[...], 0.0)
    hb = jnp.maximum(a_ref[...], 0.0)
    ss = (jnp.sum(ha * ha, axis=1, keepdims=True)
          + jnp.sum(hb * hb, axis=1, keepdims=True))
    inv = 1.0 / jnp.maximum(jnp.sqrt(ss), 1e-12)
    ha = ha * inv
    hb = hb * inv
    wh = wh_ref[...]
    o_ref[...] = (jnp.dot(ha, wh[:_H], preferred_element_type=jnp.float32)
                  + jnp.dot(hb, wh[_H:], preferred_element_type=jnp.float32)
                  + b_ref[...])


def _full(shape):
    return pl.BlockSpec(shape, lambda i: (0,) * len(shape))


def _rows(w):
    return pl.BlockSpec((_BLK, w), lambda i: (i, 0))


def _tc0(x, wp, b, ws):
    return pl.pallas_call(
        _tc0_body,
        grid=(_N // _BLK,),
        in_specs=[_rows(128), _full((128, _H)), _full((1, _H)), _full((128, _H))],
        out_specs=[_rows(_H), _rows(_H)],
        out_shape=[jax.ShapeDtypeStruct((_N, _H), jnp.float32)] * 2,
    )(x, wp, b.reshape(1, _H), ws)


def _tc1(s, a, wp, b, ws):
    return pl.pallas_call(
        _tc1_body,
        grid=(_N // _BLK,),
        in_specs=[_rows(_H), _rows(_H), _full((2 * _H, _H)), _full((1, _H)),
                  _full((2 * _H, _H))],
        out_specs=[_rows(_H), _rows(_H)],
        out_shape=[jax.ShapeDtypeStruct((_N, _H), jnp.float32)] * 2,
    )(s, a, wp, b.reshape(1, _H), ws)


def _head(s, a, wh, b):
    c = wh.shape[1]
    return pl.pallas_call(
        _head_body,
        grid=(_N // _BLK,),
        in_specs=[_rows(_H), _rows(_H), _full((2 * _H, c)), _full((1, c))],
        out_specs=_rows(c),
        out_shape=jax.ShapeDtypeStruct((_N, c), jnp.float32),
    )(s, a, wh, b.reshape(1, c))


def kernel(x, A, W_pool0, b_pool0, W_self0, W_pool1, b_pool1, W_self1,
           W_head, b_head):
    src = A[0]
    dst = A[1]
    ls, ld, cnts = _bin(src, dst)
    m0, s0 = _tc0(x, W_pool0, b_pool0, W_self0)
    agg0 = _acc(ls, ld, cnts, m0)[:_N]
    m1, s1 = _tc1(s0, agg0, W_pool1, b_pool1, W_self1)
    agg1 = _acc(ls, ld, cnts, m1)[:_N]
    return _head(s1, agg1, W_head, b_head)


# speedup vs baseline: 4.9297x; 1.1326x over previous
import functools

import jax
import jax.numpy as jnp
from jax import lax
from jax.experimental import pallas as pl
from jax.experimental.pallas import tpu as pltpu
from jax.experimental.pallas import tpu_sc as plsc

_N = 10000
_E = 320000
_H = 128
_NW = 32
_R = 320
_NPAD = _NW * _R
_G = 512
_C = 3200
_NCH = _E // _C
_F = 2048
_CAP = 160 * _F
_BUF = 5760

_SC_PARAMS = pltpu.CompilerParams(needs_layout_passes=False,
                                  use_tc_tiling_on_sc=False)
_MESH = plsc.VectorSubcoreMesh(core_axis_name="c", subcore_axis_name="s")


def _bin_body(src_hbm, dst_hbm, ls_hbm, ld_hbm, cnts_hbm,
              dc0, dc1, sc0, sc1, srcbuf, dlbuf, cbuf,
              semd0, semd1, sems0, sems1):
    wid = lax.axis_index("s") * 2 + lax.axis_index("c")
    lo = wid * _R
    rbase = wid * _CAP
    dc, sc = (dc0, dc1), (sc0, sc1)
    semd, sems = (semd0, semd1), (sems0, sems1)

    def _stage(c, k):
        pltpu.async_copy(dst_hbm.at[pl.ds(c * _C, _C)], dc[k], semd[k])
        pltpu.async_copy(src_hbm.at[pl.ds(c * _C, _C)], sc[k], sems[k])

    def _wait(k):
        pltpu.make_async_copy(dst_hbm.at[pl.ds(0, _C)], dc[k], semd[k]).wait()
        pltpu.make_async_copy(src_hbm.at[pl.ds(0, _C)], sc[k], sems[k]).wait()

    _stage(0, 0)

    def _flush(cnt, nf):
        pltpu.sync_copy(srcbuf.at[pl.ds(0, _F)],
                        ls_hbm.at[pl.ds(rbase + nf * _F, _F)])
        pltpu.sync_copy(dlbuf.at[pl.ds(0, _F)],
                        ld_hbm.at[pl.ds(rbase + nf * _F, _F)])
        for k in range(_C // 16):
            srcbuf[pl.ds(k * 16, 16)] = srcbuf[pl.ds(_F + k * 16, 16)]
            dlbuf[pl.ds(k * 16, 16)] = dlbuf[pl.ds(_F + k * 16, 16)]
        return cnt - _F, nf + 1

    def _scan_chunk(dcr, scr, cnt):
        def _quad(q, cnt):
            ss_, ds_, ms_, cs_ = [], [], [], []
            for u in range(4):
                g = q * 4 + u
                d = dcr[pl.ds(g * 16, 16)]
                s = scr[pl.ds(g * 16, 16)]
                dl = d - lo
                msk = plsc.bitcast(dl, jnp.uint32) < jnp.uint32(_R)
                pc = plsc.all_reduce_population_count(msk)
                ss_.append(s)
                ds_.append(dl)
                ms_.append(msk)
                cs_.append(pc[0])
            for u in range(4):
                plsc.store_compressed(srcbuf.at[pl.ds(cnt, 16)], ss_[u],
                                      mask=ms_[u])
                plsc.store_compressed(dlbuf.at[pl.ds(cnt, 16)], ds_[u],
                                      mask=ms_[u])
                cnt = cnt + cs_[u]
            return cnt
        return lax.fori_loop(0, _C // 64, _quad, cnt)

    def _chunk(i, carry):
        cnt, nf = carry
        for k in range(2):
            c = 2 * i + k

            @pl.when(c + 1 < _NCH)
            def _():
                _stage(c + 1, k ^ 1)
            _wait(k)
            cnt = _scan_chunk(dc[k], sc[k], cnt)
            for _ in range(2):
                cnt, nf = lax.cond(cnt >= _F, _flush,
                                   lambda c_, n_: (c_, n_), cnt, nf)
        return cnt, nf

    cnt, nf = lax.fori_loop(0, _NCH // 2, _chunk,
                            (jnp.int32(0), jnp.int32(0)))

    iota = lax.broadcasted_iota(jnp.int32, (16,), 0)
    base = (cnt // 16) * 16
    for k in range(2 * _G // 16 + 2):
        at = base + k * 16
        pos = at + iota
        mp = pos >= cnt
        sv = srcbuf[pl.ds(at, 16)]
        dv = dlbuf[pl.ds(at, 16)]
        srcbuf[pl.ds(at, 16)] = jnp.where(mp, 0, sv)
        dlbuf[pl.ds(at, 16)] = jnp.where(mp, _R, dv)

    for blk in range(2):
        pltpu.sync_copy(srcbuf.at[pl.ds(blk * _F, _F)],
                        ls_hbm.at[pl.ds(rbase + (nf + blk) * _F, _F)])
        pltpu.sync_copy(dlbuf.at[pl.ds(blk * _F, _F)],
                        ld_hbm.at[pl.ds(rbase + (nf + blk) * _F, _F)])
    cbuf[pl.ds(0, 16)] = jnp.full((16,), nf * _F + cnt, jnp.int32)
    pltpu.sync_copy(cbuf, cnts_hbm.at[pl.ds(wid * 16, 16)])


@functools.partial(
    pl.kernel,
    out_type=(jax.ShapeDtypeStruct((_NW * _CAP,), jnp.int32),
              jax.ShapeDtypeStruct((_NW * _CAP,), jnp.int32),
              jax.ShapeDtypeStruct((_NW * 16,), jnp.int32)),
    mesh=_MESH,
    compiler_params=_SC_PARAMS,
    scratch_types=[
        pltpu.VMEM((_C,), jnp.int32),
        pltpu.VMEM((_C,), jnp.int32),
        pltpu.VMEM((_C,), jnp.int32),
        pltpu.VMEM((_C,), jnp.int32),
        pltpu.VMEM((_BUF,), jnp.int32),
        pltpu.VMEM((_BUF,), jnp.int32),
        pltpu.VMEM((16,), jnp.int32),
        pltpu.SemaphoreType.DMA,
        pltpu.SemaphoreType.DMA,
        pltpu.SemaphoreType.DMA,
        pltpu.SemaphoreType.DMA,
    ],
)
def _bin(src_hbm, dst_hbm, ls_hbm, ld_hbm, cnts_hbm, *rest):
    _bin_body(src_hbm, dst_hbm, ls_hbm, ld_hbm, cnts_hbm, *rest)


def _acc_body(ls_hbm, ld_hbm, cnts_hbm, m_hbm, agg_hbm,
              acc, si0, si1, db0, db1, r0, r1, cbuf, sg0, sg1):
    wid = lax.axis_index("s") * 2 + lax.axis_index("c")
    lo = wid * _R
    rbase = wid * _CAP
    si, db, rows, sg = (si0, si1), (db0, db1), (r0, r1), (sg0, sg1)

    pltpu.sync_copy(cnts_hbm.at[pl.ds(wid * 16, 16)], cbuf)
    cnt = cbuf[pl.ds(0, 16)][0]
    nb = jnp.maximum((cnt + _G - 1) // _G, 1)

    zeros = jnp.zeros((32,), jnp.bfloat16)

    def _zero(i, carry):
        for f in range(_H // 32):
            acc[i, pl.ds(f * 32, 32)] = zeros
        return carry
    lax.fori_loop(0, _R + 1, _zero, 0)

    def _stage(b, k):
        pltpu.sync_copy(ls_hbm.at[pl.ds(rbase + b * _G, _G)], si[k])
        pltpu.sync_copy(ld_hbm.at[pl.ds(rbase + b * _G, _G)], db[k])
        for h in range(_G // 128):
            pltpu.async_copy(m_hbm.at[si[k].at[pl.ds(h * 128, 128)]],
                             rows[k].at[pl.ds(h * 128, 128), :], sg[k])

    def _wait_rows(k):
        for h in range(_G // 128):
            pltpu.make_async_copy(m_hbm.at[si[k].at[pl.ds(h * 128, 128)]],
                                  rows[k].at[pl.ds(h * 128, 128), :],
                                  sg[k]).wait()

    _stage(0, 0)

    def _outer(i, carry):
        for k in range(2):
            b = 2 * i + k

            @pl.when(b < nb)
            def _():
                @pl.when(b + 1 < nb)
                def _():
                    _stage(b + 1, k ^ 1)
                _wait_rows(k)

                def _block(jb, carry2):
                    dlv = db[k][pl.ds(jb * 16, 16)]
                    for i16 in range(16):
                        j = jb * 16 + i16
                        dl = dlv[i16]
                        av = [acc[dl, pl.ds(f * 32, 32)]
                              for f in range(_H // 32)]
                        rv = [plsc.bitcast(rows[k][j, pl.ds(f * 16, 16)],
                                           jnp.bfloat16)
                              for f in range(_H // 32)]
                        for f in range(_H // 32):
                            acc[dl, pl.ds(f * 32, 32)] = jnp.maximum(av[f],
                                                                     rv[f])
                    return carry2
                lax.fori_loop(0, _G // 16, _block, 0)
        return carry

    lax.fori_loop(0, (nb + 1) // 2, _outer, 0)
    pltpu.sync_copy(acc.at[pl.ds(0, _R), :], agg_hbm.at[pl.ds(lo, _R), :])


@functools.partial(
    pl.kernel,
    out_type=jax.ShapeDtypeStruct((_NPAD, _H), jnp.bfloat16),
    mesh=_MESH,
    compiler_params=_SC_PARAMS,
    scratch_types=[
        pltpu.VMEM((_R + 1, _H), jnp.bfloat16),
        pltpu.VMEM((_G,), jnp.int32),
        pltpu.VMEM((_G,), jnp.int32),
        pltpu.VMEM((_G,), jnp.int32),
        pltpu.VMEM((_G,), jnp.int32),
        pltpu.VMEM((_G, _H // 2), jnp.int32),
        pltpu.VMEM((_G, _H // 2), jnp.int32),
        pltpu.VMEM((16,), jnp.int32),
        pltpu.SemaphoreType.DMA,
        pltpu.SemaphoreType.DMA,
    ],
)
def _acc(ls_hbm, ld_hbm, cnts_hbm, m_hbm, agg_hbm, *rest):
    _acc_body(ls_hbm, ld_hbm, cnts_hbm, m_hbm, agg_hbm, *rest)


_BLK = 2000


def _tc0_body(x_ref, wp_ref, b_ref, ws_ref, m_ref, s_ref):
    x = x_ref[...]
    m = jnp.dot(x, wp_ref[...], preferred_element_type=jnp.float32) + b_ref[...]
    m_ref[...] = jnp.maximum(m, 0.0).astype(jnp.bfloat16)
    s_ref[...] = jnp.dot(x, ws_ref[...], preferred_element_type=jnp.float32)


def _tc1_body(s_ref, a_ref, wp_ref, b_ref, ws_ref, m_ref, so_ref):
    ha = jnp.maximum(s_ref[...], 0.0)
    hb = jnp.maximum(a_ref[...].astype(jnp.float32), 0.0)
    ss = (jnp.sum(ha * ha, axis=1, keepdims=True)
          + jnp.sum(hb * hb, axis=1, keepdims=True))
    inv = 1.0 / jnp.maximum(jnp.sqrt(ss), 1e-12)
    ha = ha * inv
    hb = hb * inv
    wp = wp_ref[...]
    m = (jnp.dot(ha, wp[:_H], preferred_element_type=jnp.float32)
         + jnp.dot(hb, wp[_H:], preferred_element_type=jnp.float32)
         + b_ref[...])
    m_ref[...] = jnp.maximum(m, 0.0).astype(jnp.bfloat16)
    ws = ws_ref[...]
    so_ref[...] = (jnp.dot(ha, ws[:_H], preferred_element_type=jnp.float32)
                   + jnp.dot(hb, ws[_H:], preferred_element_type=jnp.float32))


def _head_body(s_ref, a_ref, wh_ref, b_ref, o_ref):
    ha = jnp.maximum(s_ref[...], 0.0)
    hb = jnp.maximum(a_ref[...].astype(jnp.float32), 0.0)
    ss = (jnp.sum(ha * ha, axis=1, keepdims=True)
          + jnp.sum(hb * hb, axis=1, keepdims=True))
    inv = 1.0 / jnp.maximum(jnp.sqrt(ss), 1e-12)
    ha = ha * inv
    hb = hb * inv
    wh = wh_ref[...]
    o_ref[...] = (jnp.dot(ha, wh[:_H], preferred_element_type=jnp.float32)
                  + jnp.dot(hb, wh[_H:], preferred_element_type=jnp.float32)
                  + b_ref[...])


def _full(shape):
    return pl.BlockSpec(shape, lambda i: (0,) * len(shape))


def _rows(w):
    return pl.BlockSpec((_BLK, w), lambda i: (i, 0))


def _tc0(x, wp, b, ws):
    return pl.pallas_call(
        _tc0_body,
        grid=(_N // _BLK,),
        in_specs=[_rows(128), _full((128, _H)), _full((1, _H)), _full((128, _H))],
        out_specs=[_rows(_H), _rows(_H)],
        out_shape=[jax.ShapeDtypeStruct((_N, _H), jnp.bfloat16),
                   jax.ShapeDtypeStruct((_N, _H), jnp.float32)],
    )(x, wp, b.reshape(1, _H), ws)


def _tc1(s, a, wp, b, ws):
    return pl.pallas_call(
        _tc1_body,
        grid=(_N // _BLK,),
        in_specs=[_rows(_H), _rows(_H), _full((2 * _H, _H)), _full((1, _H)),
                  _full((2 * _H, _H))],
        out_specs=[_rows(_H), _rows(_H)],
        out_shape=[jax.ShapeDtypeStruct((_N, _H), jnp.bfloat16),
                   jax.ShapeDtypeStruct((_N, _H), jnp.float32)],
    )(s, a, wp, b.reshape(1, _H), ws)


def _head(s, a, wh, b):
    c = wh.shape[1]
    return pl.pallas_call(
        _head_body,
        grid=(_N // _BLK,),
        in_specs=[_rows(_H), _rows(_H), _full((2 * _H, c)), _full((1, c))],
        out_specs=_rows(c),
        out_shape=jax.ShapeDtypeStruct((_N, c), jnp.float32),
    )(s, a, wh, b.reshape(1, c))


def kernel(x, A, W_pool0, b_pool0, W_self0, W_pool1, b_pool1, W_self1,
           W_head, b_head):
    src = A[0]
    dst = A[1]
    ls, ld, cnts = _bin(src, dst)
    m0, s0 = _tc0(x, W_pool0, b_pool0, W_self0)
    m0i = lax.bitcast_convert_type(m0.reshape(_N, _H // 2, 2), jnp.int32)
    agg0 = _acc(ls, ld, cnts, m0i)[:_N]
    m1, s1 = _tc1(s0, agg0, W_pool1, b_pool1, W_self1)
    m1i = lax.bitcast_convert_type(m1.reshape(_N, _H // 2, 2), jnp.int32)
    agg1 = _acc(ls, ld, cnts, m1i)[:_N]
    return _head(s1, agg1, W_head, b_head)


# speedup vs baseline: 5.1510x; 1.0449x over previous
import functools

import jax
import jax.numpy as jnp
from jax import lax
from jax.experimental import pallas as pl
from jax.experimental.pallas import tpu as pltpu
from jax.experimental.pallas import tpu_sc as plsc

_N = 10000
_E = 320000
_H = 128
_NW = 32
_R = 320
_NPAD = _NW * _R
_G = 512
_C = 3200
_NCH = _E // _C
_F = 2048
_CAP = 160 * _F
_BUF = 5760

_SC_PARAMS = pltpu.CompilerParams(needs_layout_passes=False,
                                  use_tc_tiling_on_sc=False)
_MESH = plsc.VectorSubcoreMesh(core_axis_name="c", subcore_axis_name="s")


def _bin_body(src_hbm, dst_hbm, ls_hbm, ld_hbm, cnts_hbm,
              dc0, dc1, sc0, sc1, srcbuf, dlbuf, cbuf,
              semd0, semd1, sems0, sems1):
    wid = lax.axis_index("s") * 2 + lax.axis_index("c")
    lo = wid * _R
    rbase = wid * _CAP
    dc, sc = (dc0, dc1), (sc0, sc1)
    semd, sems = (semd0, semd1), (sems0, sems1)

    def _stage(c, k):
        pltpu.async_copy(dst_hbm.at[pl.ds(c * _C, _C)], dc[k], semd[k])
        pltpu.async_copy(src_hbm.at[pl.ds(c * _C, _C)], sc[k], sems[k])

    def _wait(k):
        pltpu.make_async_copy(dst_hbm.at[pl.ds(0, _C)], dc[k], semd[k]).wait()
        pltpu.make_async_copy(src_hbm.at[pl.ds(0, _C)], sc[k], sems[k]).wait()

    _stage(0, 0)

    def _flush(cnt, nf):
        pltpu.sync_copy(srcbuf.at[pl.ds(0, _F)],
                        ls_hbm.at[pl.ds(rbase + nf * _F, _F)])
        pltpu.sync_copy(dlbuf.at[pl.ds(0, _F)],
                        ld_hbm.at[pl.ds(rbase + nf * _F, _F)])
        for k in range(_C // 16):
            srcbuf[pl.ds(k * 16, 16)] = srcbuf[pl.ds(_F + k * 16, 16)]
            dlbuf[pl.ds(k * 16, 16)] = dlbuf[pl.ds(_F + k * 16, 16)]
        return cnt - _F, nf + 1

    def _scan_chunk(dcr, scr, cnt):
        def _quad(q, cnt):
            ss_, ds_, ms_, cs_ = [], [], [], []
            for u in range(4):
                g = q * 4 + u
                d = dcr[pl.ds(g * 16, 16)]
                s = scr[pl.ds(g * 16, 16)]
                dl = d - lo
                msk = plsc.bitcast(dl, jnp.uint32) < jnp.uint32(_R)
                pc = plsc.all_reduce_population_count(msk)
                ss_.append(s)
                ds_.append(dl)
                ms_.append(msk)
                cs_.append(pc[0])
            for u in range(4):
                plsc.store_compressed(srcbuf.at[pl.ds(cnt, 16)], ss_[u],
                                      mask=ms_[u])
                plsc.store_compressed(dlbuf.at[pl.ds(cnt, 16)], ds_[u],
                                      mask=ms_[u])
                cnt = cnt + cs_[u]
            return cnt
        return lax.fori_loop(0, _C // 64, _quad, cnt)

    def _chunk(i, carry):
        cnt, nf = carry
        for k in range(2):
            c = 2 * i + k

            @pl.when(c + 1 < _NCH)
            def _():
                _stage(c + 1, k ^ 1)
            _wait(k)
            cnt = _scan_chunk(dc[k], sc[k], cnt)
            for _ in range(2):
                cnt, nf = lax.cond(cnt >= _F, _flush,
                                   lambda c_, n_: (c_, n_), cnt, nf)
        return cnt, nf

    cnt, nf = lax.fori_loop(0, _NCH // 2, _chunk,
                            (jnp.int32(0), jnp.int32(0)))

    iota = lax.broadcasted_iota(jnp.int32, (16,), 0)
    base = (cnt // 16) * 16
    for k in range(2 * _G // 16 + 2):
        at = base + k * 16
        pos = at + iota
        mp = pos >= cnt
        sv = srcbuf[pl.ds(at, 16)]
        dv = dlbuf[pl.ds(at, 16)]
        srcbuf[pl.ds(at, 16)] = jnp.where(mp, 0, sv)
        dlbuf[pl.ds(at, 16)] = jnp.where(mp, _R, dv)

    for blk in range(2):
        pltpu.sync_copy(srcbuf.at[pl.ds(blk * _F, _F)],
                        ls_hbm.at[pl.ds(rbase + (nf + blk) * _F, _F)])
        pltpu.sync_copy(dlbuf.at[pl.ds(blk * _F, _F)],
                        ld_hbm.at[pl.ds(rbase + (nf + blk) * _F, _F)])
    cbuf[pl.ds(0, 16)] = jnp.full((16,), nf * _F + cnt, jnp.int32)
    pltpu.sync_copy(cbuf, cnts_hbm.at[pl.ds(wid * 16, 16)])


@functools.partial(
    pl.kernel,
    out_type=(jax.ShapeDtypeStruct((_NW * _CAP,), jnp.int32),
              jax.ShapeDtypeStruct((_NW * _CAP,), jnp.int32),
              jax.ShapeDtypeStruct((_NW * 16,), jnp.int32)),
    mesh=_MESH,
    compiler_params=_SC_PARAMS,
    scratch_types=[
        pltpu.VMEM((_C,), jnp.int32),
        pltpu.VMEM((_C,), jnp.int32),
        pltpu.VMEM((_C,), jnp.int32),
        pltpu.VMEM((_C,), jnp.int32),
        pltpu.VMEM((_BUF,), jnp.int32),
        pltpu.VMEM((_BUF,), jnp.int32),
        pltpu.VMEM((16,), jnp.int32),
        pltpu.SemaphoreType.DMA,
        pltpu.SemaphoreType.DMA,
        pltpu.SemaphoreType.DMA,
        pltpu.SemaphoreType.DMA,
    ],
)
def _bin(src_hbm, dst_hbm, ls_hbm, ld_hbm, cnts_hbm, *rest):
    _bin_body(src_hbm, dst_hbm, ls_hbm, ld_hbm, cnts_hbm, *rest)


_BPS = 32
_LMAX = _BPS * _G


def _acc_body(ls_hbm, ld_hbm, cnts_hbm, m_hbm, agg_hbm,
              acc, sall, dall, r0, r1, cbuf, sg0, sg1):
    wid = lax.axis_index("s") * 2 + lax.axis_index("c")
    lo = wid * _R
    rbase = wid * _CAP
    rows, sg = (r0, r1), (sg0, sg1)

    pltpu.sync_copy(cnts_hbm.at[pl.ds(wid * 16, 16)], cbuf)
    cnt = cbuf[pl.ds(0, 16)][0]
    nb = jnp.maximum((cnt + _G - 1) // _G, 1)
    nsb = (nb + _BPS - 1) // _BPS

    zeros = jnp.zeros((32,), jnp.bfloat16)

    def _zero(i, carry):
        for f in range(_H // 32):
            acc[i, pl.ds(f * 32, 32)] = zeros
        return carry
    lax.fori_loop(0, _R + 1, _zero, 0)

    def _fire(bl, k):
        for h in range(_G // 128):
            pltpu.async_copy(m_hbm.at[sall.at[pl.ds(bl * _G + h * 128, 128)]],
                             rows[k].at[pl.ds(h * 128, 128), :], sg[k])

    def _wait_rows(k):
        for h in range(_G // 128):
            pltpu.make_async_copy(m_hbm.at[sall.at[pl.ds(h * 128, 128)]],
                                  rows[k].at[pl.ds(h * 128, 128), :],
                                  sg[k]).wait()

    def _super(s, carry):
        pltpu.sync_copy(ls_hbm.at[pl.ds(rbase + s * _LMAX, _LMAX)], sall)
        pltpu.sync_copy(ld_hbm.at[pl.ds(rbase + s * _LMAX, _LMAX)], dall)
        nbs = jnp.minimum(nb - s * _BPS, _BPS)
        _fire(0, 0)

        def _outer(i, c2):
            for k in range(2):
                bl = 2 * i + k

                @pl.when(bl < nbs)
                def _():
                    @pl.when(bl + 1 < nbs)
                    def _():
                        _fire(bl + 1, k ^ 1)
                    _wait_rows(k)

                    def _block(jb, c3):
                        dlv = dall[pl.ds(bl * _G + jb * 16, 16)]
                        for i16 in range(16):
                            j = jb * 16 + i16
                            dl = dlv[i16]
                            av = [acc[dl, pl.ds(f * 32, 32)]
                                  for f in range(_H // 32)]
                            rv = [plsc.bitcast(rows[k][j, pl.ds(f * 16, 16)],
                                               jnp.bfloat16)
                                  for f in range(_H // 32)]
                            for f in range(_H // 32):
                                acc[dl, pl.ds(f * 32, 32)] = jnp.maximum(
                                    av[f], rv[f])
                        return c3
                    lax.fori_loop(0, _G // 16, _block, 0)
            return c2
        lax.fori_loop(0, _BPS // 2, _outer, 0)
        return carry

    lax.fori_loop(0, nsb, _super, 0)
    pltpu.sync_copy(acc.at[pl.ds(0, _R), :], agg_hbm.at[pl.ds(lo, _R), :])


@functools.partial(
    pl.kernel,
    out_type=jax.ShapeDtypeStruct((_NPAD, _H), jnp.bfloat16),
    mesh=_MESH,
    compiler_params=_SC_PARAMS,
    scratch_types=[
        pltpu.VMEM((_R + 1, _H), jnp.bfloat16),
        pltpu.VMEM((_LMAX,), jnp.int32),
        pltpu.VMEM((_LMAX,), jnp.int32),
        pltpu.VMEM((_G, _H // 2), jnp.int32),
        pltpu.VMEM((_G, _H // 2), jnp.int32),
        pltpu.VMEM((16,), jnp.int32),
        pltpu.SemaphoreType.DMA,
        pltpu.SemaphoreType.DMA,
    ],
)
def _acc(ls_hbm, ld_hbm, cnts_hbm, m_hbm, agg_hbm, *rest):
    _acc_body(ls_hbm, ld_hbm, cnts_hbm, m_hbm, agg_hbm, *rest)


_BLK = 2000


def _tc0_body(x_ref, wp_ref, b_ref, ws_ref, m_ref, s_ref):
    x = x_ref[...]
    m = jnp.dot(x, wp_ref[...], preferred_element_type=jnp.float32) + b_ref[...]
    m_ref[...] = jnp.maximum(m, 0.0).astype(jnp.bfloat16)
    s_ref[...] = jnp.dot(x, ws_ref[...], preferred_element_type=jnp.float32)


def _tc1_body(s_ref, a_ref, wp_ref, b_ref, ws_ref, m_ref, so_ref):
    ha = jnp.maximum(s_ref[...], 0.0)
    hb = jnp.maximum(a_ref[...].astype(jnp.float32), 0.0)
    ss = (jnp.sum(ha * ha, axis=1, keepdims=True)
          + jnp.sum(hb * hb, axis=1, keepdims=True))
    inv = 1.0 / jnp.maximum(jnp.sqrt(ss), 1e-12)
    ha = ha * inv
    hb = hb * inv
    wp = wp_ref[...]
    m = (jnp.dot(ha, wp[:_H], preferred_element_type=jnp.float32)
         + jnp.dot(hb, wp[_H:], preferred_element_type=jnp.float32)
         + b_ref[...])
    m_ref[...] = jnp.maximum(m, 0.0).astype(jnp.bfloat16)
    ws = ws_ref[...]
    so_ref[...] = (jnp.dot(ha, ws[:_H], preferred_element_type=jnp.float32)
                   + jnp.dot(hb, ws[_H:], preferred_element_type=jnp.float32))


def _head_body(s_ref, a_ref, wh_ref, b_ref, o_ref):
    ha = jnp.maximum(s_ref[...], 0.0)
    hb = jnp.maximum(a_ref[...].astype(jnp.float32), 0.0)
    ss = (jnp.sum(ha * ha, axis=1, keepdims=True)
          + jnp.sum(hb * hb, axis=1, keepdims=True))
    inv = 1.0 / jnp.maximum(jnp.sqrt(ss), 1e-12)
    ha = ha * inv
    hb = hb * inv
    wh = wh_ref[...]
    o_ref[...] = (jnp.dot(ha, wh[:_H], preferred_element_type=jnp.float32)
                  + jnp.dot(hb, wh[_H:], preferred_element_type=jnp.float32)
                  + b_ref[...])


def _full(shape):
    return pl.BlockSpec(shape, lambda i: (0,) * len(shape))


def _rows(w):
    return pl.BlockSpec((_BLK, w), lambda i: (i, 0))


def _tc0(x, wp, b, ws):
    return pl.pallas_call(
        _tc0_body,
        grid=(_N // _BLK,),
        in_specs=[_rows(128), _full((128, _H)), _full((1, _H)), _full((128, _H))],
        out_specs=[_rows(_H), _rows(_H)],
        out_shape=[jax.ShapeDtypeStruct((_N, _H), jnp.bfloat16),
                   jax.ShapeDtypeStruct((_N, _H), jnp.float32)],
    )(x, wp, b.reshape(1, _H), ws)


def _tc1(s, a, wp, b, ws):
    return pl.pallas_call(
        _tc1_body,
        grid=(_N // _BLK,),
        in_specs=[_rows(_H), _rows(_H), _full((2 * _H, _H)), _full((1, _H)),
                  _full((2 * _H, _H))],
        out_specs=[_rows(_H), _rows(_H)],
        out_shape=[jax.ShapeDtypeStruct((_N, _H), jnp.bfloat16),
                   jax.ShapeDtypeStruct((_N, _H), jnp.float32)],
    )(s, a, wp, b.reshape(1, _H), ws)


def _head(s, a, wh, b):
    c = wh.shape[1]
    return pl.pallas_call(
        _head_body,
        grid=(_N // _BLK,),
        in_specs=[_rows(_H), _rows(_H), _full((2 * _H, c)), _full((1, c))],
        out_specs=_rows(c),
        out_shape=jax.ShapeDtypeStruct((_N, c), jnp.float32),
    )(s, a, wh, b.reshape(1, c))


def kernel(x, A, W_pool0, b_pool0, W_self0, W_pool1, b_pool1, W_self1,
           W_head, b_head):
    src = A[0]
    dst = A[1]
    ls, ld, cnts = _bin(src, dst)
    m0, s0 = _tc0(x, W_pool0, b_pool0, W_self0)
    m0i = lax.bitcast_convert_type(m0.reshape(_N, _H // 2, 2), jnp.int32)
    agg0 = _acc(ls, ld, cnts, m0i)[:_N]
    m1, s1 = _tc1(s0, agg0, W_pool1, b_pool1, W_self1)
    m1i = lax.bitcast_convert_type(m1.reshape(_N, _H // 2, 2), jnp.int32)
    agg1 = _acc(ls, ld, cnts, m1i)[:_N]
    return _head(s1, agg1, W_head, b_head)
